# Initial kernel scaffold; baseline (speedup 1.0000x reference)
import jax
import jax.numpy as jnp
from jax.experimental import pallas as pl


def kernel(x, edge_index, lin0_w, lin0_b, conv_w, lin1_w, lin1_b):
    raise NotImplementedError("write your pallas kernel here")


# speedup vs baseline: 4.4851x; 4.4851x over previous
import functools

import numpy as np
import jax
import jax.numpy as jnp
from jax import lax
from jax.experimental import pallas as pl
from jax.experimental.pallas import tpu as pltpu
from jax.experimental.pallas import tpu_sc as plsc

N = 10000
E = 320000
D = 128
H = 64
C = 40
L = 4
ALPHA = 0.1
THETA = 0.5
MIN_NORM = 1e-15
EPS = 4e-3
MAXNORM = 1.0 - EPS

NC = 2
NS = 16
NW = NC * NS
CHUNK = 128
NCHUNKS = 80
E_PAD = NW * NCHUNKS * CHUNK
N_PAD = 10240
ROWS_PER_TILE = N_PAD // NS
CPAD = 128


def _spmm_body(h_hbm, srcp_hbm, dstp_hbm, zeros_hbm, agg_hbm,
               src_v, dst_v, rows_v, agg_sh, sem0, sem1):
    cid = lax.axis_index("c")
    sid = lax.axis_index("s")
    wid = cid * NS + sid

    pltpu.sync_copy(srcp_hbm.at[wid], src_v)
    pltpu.sync_copy(dstp_hbm.at[wid], dst_v)
    pltpu.sync_copy(zeros_hbm, agg_sh.at[pl.ds(sid * ROWS_PER_TILE, ROWS_PER_TILE)])
    plsc.subcore_barrier()

    def body(it, carry):
        j0 = it * 2
        cp0 = pltpu.async_copy(h_hbm.at[src_v.at[j0]], rows_v.at[0], sem0)
        cp1 = pltpu.async_copy(h_hbm.at[src_v.at[j0 + 1]], rows_v.at[1], sem1)
        cp0.wait()
        pltpu.sync_copy(rows_v.at[0], agg_sh.at[dst_v.at[j0]], add=True)
        cp1.wait()
        pltpu.sync_copy(rows_v.at[1], agg_sh.at[dst_v.at[j0 + 1]], add=True)
        return carry

    lax.fori_loop(0, NCHUNKS // 2, body, 0)
    plsc.subcore_barrier()
    out_base = cid * N_PAD + sid * ROWS_PER_TILE
    pltpu.sync_copy(agg_sh.at[pl.ds(sid * ROWS_PER_TILE, ROWS_PER_TILE)],
                    agg_hbm.at[pl.ds(out_base, ROWS_PER_TILE)])


_spmm = functools.partial(
    pl.kernel,
    out_type=jax.ShapeDtypeStruct((NC * N_PAD, H), jnp.float32),
    mesh=plsc.VectorSubcoreMesh(core_axis_name="c", subcore_axis_name="s"),
    scratch_types=[
        pltpu.VMEM((NCHUNKS, CHUNK), jnp.int32),
        pltpu.VMEM((NCHUNKS, CHUNK), jnp.int32),
        pltpu.VMEM((2, CHUNK, H), jnp.float32),
        pltpu.VMEM_SHARED((N_PAD, H), jnp.float32),
        pltpu.SemaphoreType.DMA,
        pltpu.SemaphoreType.DMA,
    ],
    compiler_params=pltpu.CompilerParams(use_tc_tiling_on_sc=False),
)(_spmm_body)


BN0 = 2000


def _h0_body(x_ref, w_ref, b_ref, o_ref):
    o_ref[...] = jnp.maximum(
        jnp.dot(x_ref[...], w_ref[...], preferred_element_type=jnp.float32)
        + b_ref[...], 0.0)


def _h0_call(x, w, b):
    return pl.pallas_call(
        _h0_body,
        grid=(N // BN0,),
        in_specs=[
            pl.BlockSpec((BN0, D), lambda i: (i, 0)),
            pl.BlockSpec((D, H), lambda i: (0, 0)),
            pl.BlockSpec((1, H), lambda i: (0, 0)),
        ],
        out_specs=pl.BlockSpec((BN0, H), lambda i: (i, 0)),
        out_shape=jax.ShapeDtypeStruct((N, H), jnp.float32),
    )(x, w, b)


def _layer_body(beta, a0_ref, a1_ref, h0_ref, w_ref, o_ref):
    agg = a0_ref[...] + a1_ref[...]
    hh = (1.0 - ALPHA) * agg + ALPHA * h0_ref[...]
    out = (1.0 - beta) * hh + beta * jnp.dot(
        hh, w_ref[...], preferred_element_type=jnp.float32)
    o_ref[...] = jnp.maximum(out, 0.0)


def _layer_call(beta, a0, a1, h0, w):
    return pl.pallas_call(
        functools.partial(_layer_body, beta),
        grid=(N // BN0,),
        in_specs=[
            pl.BlockSpec((BN0, H), lambda i: (i, 0)),
            pl.BlockSpec((BN0, H), lambda i: (i, 0)),
            pl.BlockSpec((BN0, H), lambda i: (i, 0)),
            pl.BlockSpec((H, H), lambda i: (0, 0)),
        ],
        out_specs=pl.BlockSpec((BN0, H), lambda i: (i, 0)),
        out_shape=jax.ShapeDtypeStruct((N, H), jnp.float32),
    )(a0, a1, h0, w)


BNF = 400


def _final_body(h_ref, w4_ref, b_ref, o_ref):
    h = h_ref[...]
    t = jnp.dot(h, w4_ref[...], preferred_element_type=jnp.float32)
    acc = jnp.zeros((BNF, CPAD), jnp.float32)
    for i in range(H):
        acc = acc + h[:, i:i + 1] * t[:, i * CPAD:(i + 1) * CPAD]
    q = acc[:, :C]
    s1 = jnp.sum(h * h, axis=1, keepdims=True)
    nrm = jnp.maximum(s1, MIN_NORM)
    f1 = jnp.where(nrm > MAXNORM, MAXNORM / nrm, 1.0)
    pn = jnp.maximum(nrm * f1, MIN_NORM)
    pc = jnp.minimum(pn, 1.0 - 1e-7)
    art = 0.5 * (jnp.log1p(pc) - jnp.log1p(-pc))
    scale = f1 * art / pn
    y = scale * q + b_ref[...]
    un = jnp.maximum(jnp.sqrt(jnp.sum(y * y, axis=1, keepdims=True)), MIN_NORM)
    res = jnp.tanh(un) * y / un
    rn = jnp.maximum(jnp.sqrt(jnp.sum(res * res, axis=1, keepdims=True)), MIN_NORM)
    res = jnp.where(rn > MAXNORM, res / rn * MAXNORM, res)
    m = jnp.max(res, axis=1, keepdims=True)
    z = res - m
    o_ref[...] = z - jnp.log(jnp.sum(jnp.exp(z), axis=1, keepdims=True))


def _final_call(h, w4, b):
    return pl.pallas_call(
        _final_body,
        grid=(N // BNF,),
        in_specs=[
            pl.BlockSpec((BNF, H), lambda i: (i, 0)),
            pl.BlockSpec((H, H * CPAD), lambda i: (0, 0)),
            pl.BlockSpec((1, C), lambda i: (0, 0)),
        ],
        out_specs=pl.BlockSpec((BNF, C), lambda i: (i, 0)),
        out_shape=jax.ShapeDtypeStruct((N, C), jnp.float32),
    )(h, w4, b)


def kernel(x, edge_index, lin0_w, lin0_b, conv_w, lin1_w, lin1_b):
    x = x.astype(jnp.float32)
    ei = edge_index.astype(jnp.int32)
    src_p = jnp.concatenate(
        [ei[0], jnp.zeros((E_PAD - E,), jnp.int32)]).reshape(NW, NCHUNKS, CHUNK)
    dst_p = jnp.concatenate(
        [ei[1], jnp.full((E_PAD - E,), N, jnp.int32)]).reshape(NW, NCHUNKS, CHUNK)
    zeros_slab = jnp.zeros((ROWS_PER_TILE, H), jnp.float32)

    h0 = _h0_call(x, lin0_w, lin0_b.reshape(1, H))
    h = h0
    for l in range(L):
        beta = float(np.log(THETA / (l + 1) + 1.0))
        agg2 = _spmm(h, src_p, dst_p, zeros_slab)
        h = _layer_call(beta, agg2[:N], agg2[N_PAD:N_PAD + N], h0, conv_w[l])

    w4 = jnp.zeros((H, H, CPAD), jnp.float32).at[:, :, :C].set(
        lin1_w.reshape(H, H, C).transpose(1, 0, 2)).reshape(H, H * CPAD)
    return _final_call(h, w4, lin1_b.reshape(1, C))


# speedup vs baseline: 8.6710x; 1.9333x over previous
import functools

import numpy as np
import jax
import jax.numpy as jnp
from jax import lax
from jax.experimental import pallas as pl
from jax.experimental.pallas import tpu as pltpu
from jax.experimental.pallas import tpu_sc as plsc

N = 10000
E = 320000
D = 128
H = 64
C = 40
L = 4
ALPHA = 0.1
THETA = 0.5
MIN_NORM = 1e-15
EPS = 4e-3
MAXNORM = 1.0 - EPS

NC = 2
NS = 16
NW = NC * NS
CHUNK = 128
NCHUNKS = 80
E_PAD = NW * NCHUNKS * CHUNK
N_PAD = 10240
ROWS_PER_TILE = N_PAD // NS
CPAD = 128


def _spmm_body(h_hbm, srcp_hbm, dstp_hbm, zeros_hbm, agg_hbm,
               src_v, dst_v, rows_v, agg_sh, sem0, sem1):
    cid = lax.axis_index("c")
    sid = lax.axis_index("s")
    wid = cid * NS + sid

    pltpu.sync_copy(srcp_hbm.at[wid], src_v)
    pltpu.sync_copy(dstp_hbm.at[wid], dst_v)
    pltpu.sync_copy(zeros_hbm, agg_sh.at[pl.ds(sid * ROWS_PER_TILE, ROWS_PER_TILE)])
    plsc.subcore_barrier()

    def body(it, carry):
        j0 = it * 2
        cp0 = pltpu.async_copy(h_hbm.at[src_v.at[j0]], rows_v.at[0], sem0)
        cp1 = pltpu.async_copy(h_hbm.at[src_v.at[j0 + 1]], rows_v.at[1], sem1)
        cp0.wait()
        pltpu.sync_copy(rows_v.at[0], agg_sh.at[dst_v.at[j0]], add=True)
        cp1.wait()
        pltpu.sync_copy(rows_v.at[1], agg_sh.at[dst_v.at[j0 + 1]], add=True)
        return carry

    lax.fori_loop(0, NCHUNKS // 2, body, 0)
    plsc.subcore_barrier()
    out_base = cid * N_PAD + sid * ROWS_PER_TILE
    pltpu.sync_copy(agg_sh.at[pl.ds(sid * ROWS_PER_TILE, ROWS_PER_TILE)],
                    agg_hbm.at[pl.ds(out_base, ROWS_PER_TILE)])


_spmm = functools.partial(
    pl.kernel,
    out_type=jax.ShapeDtypeStruct((NC * N_PAD, H), jnp.float32),
    mesh=plsc.VectorSubcoreMesh(core_axis_name="c", subcore_axis_name="s"),
    scratch_types=[
        pltpu.VMEM((NCHUNKS, CHUNK), jnp.int32),
        pltpu.VMEM((NCHUNKS, CHUNK), jnp.int32),
        pltpu.VMEM((2, CHUNK, H), jnp.float32),
        pltpu.VMEM_SHARED((N_PAD, H), jnp.float32),
        pltpu.SemaphoreType.DMA,
        pltpu.SemaphoreType.DMA,
    ],
    compiler_params=pltpu.CompilerParams(use_tc_tiling_on_sc=False),
)(_spmm_body)


BN0 = 2000


def _h0_body(x_ref, w_ref, b_ref, o_ref):
    o_ref[...] = jnp.maximum(
        jnp.dot(x_ref[...], w_ref[...], preferred_element_type=jnp.float32)
        + b_ref[...], 0.0)


def _h0_call(x, w, b):
    return pl.pallas_call(
        _h0_body,
        grid=(N // BN0,),
        in_specs=[
            pl.BlockSpec((BN0, D), lambda i: (i, 0)),
            pl.BlockSpec((D, H), lambda i: (0, 0)),
            pl.BlockSpec((1, H), lambda i: (0, 0)),
        ],
        out_specs=pl.BlockSpec((BN0, H), lambda i: (i, 0)),
        out_shape=jax.ShapeDtypeStruct((N, H), jnp.float32),
    )(x, w, b)


def _layer_body(beta, a0_ref, a1_ref, h0_ref, w_ref, o_ref):
    agg = a0_ref[...] + a1_ref[...]
    hh = (1.0 - ALPHA) * agg + ALPHA * h0_ref[...]
    out = (1.0 - beta) * hh + beta * jnp.dot(
        hh, w_ref[...], preferred_element_type=jnp.float32)
    o_ref[...] = jnp.maximum(out, 0.0)


def _layer_call(beta, a0, a1, h0, w):
    return pl.pallas_call(
        functools.partial(_layer_body, beta),
        grid=(N // BN0,),
        in_specs=[
            pl.BlockSpec((BN0, H), lambda i: (i, 0)),
            pl.BlockSpec((BN0, H), lambda i: (i, 0)),
            pl.BlockSpec((BN0, H), lambda i: (i, 0)),
            pl.BlockSpec((H, H), lambda i: (0, 0)),
        ],
        out_specs=pl.BlockSpec((BN0, H), lambda i: (i, 0)),
        out_shape=jax.ShapeDtypeStruct((N, H), jnp.float32),
    )(a0, a1, h0, w)


BNF = 400


def _final_body(h_ref, w4_ref, b_ref, o_ref):
    h = h_ref[...]
    t = jnp.dot(h, w4_ref[...], preferred_element_type=jnp.float32)
    acc = jnp.zeros((BNF, CPAD), jnp.float32)
    for i in range(H):
        acc = acc + h[:, i:i + 1] * t[:, i * CPAD:(i + 1) * CPAD]
    q = acc[:, :C]
    s1 = jnp.sum(h * h, axis=1, keepdims=True)
    nrm = jnp.maximum(s1, MIN_NORM)
    f1 = jnp.where(nrm > MAXNORM, MAXNORM / nrm, 1.0)
    pn = jnp.maximum(nrm * f1, MIN_NORM)
    pc = jnp.minimum(pn, 1.0 - 1e-7)
    art = 0.5 * (jnp.log1p(pc) - jnp.log1p(-pc))
    scale = f1 * art / pn
    y = scale * q + b_ref[...]
    un = jnp.maximum(jnp.sqrt(jnp.sum(y * y, axis=1, keepdims=True)), MIN_NORM)
    res = jnp.tanh(un) * y / un
    rn = jnp.maximum(jnp.sqrt(jnp.sum(res * res, axis=1, keepdims=True)), MIN_NORM)
    res = jnp.where(rn > MAXNORM, res / rn * MAXNORM, res)
    m = jnp.max(res, axis=1, keepdims=True)
    z = res - m
    o_ref[...] = z - jnp.log(jnp.sum(jnp.exp(z), axis=1, keepdims=True))


def _final_call(h, w4, b):
    return pl.pallas_call(
        _final_body,
        grid=(N // BNF,),
        in_specs=[
            pl.BlockSpec((BNF, H), lambda i: (i, 0)),
            pl.BlockSpec((H, H * CPAD), lambda i: (0, 0)),
            pl.BlockSpec((1, C), lambda i: (0, 0)),
        ],
        out_specs=pl.BlockSpec((BNF, C), lambda i: (i, 0)),
        out_shape=jax.ShapeDtypeStruct((N, C), jnp.float32),
    )(h, w4, b)


def kernel(x, edge_index, lin0_w, lin0_b, conv_w, lin1_w, lin1_b):
    x = x.astype(jnp.float32)
    ei = edge_index.astype(jnp.int32)
    lane = jnp.arange(E_PAD - E, dtype=jnp.int32) % CHUNK
    src_p = jnp.concatenate([ei[0], lane]).reshape(
        NCHUNKS, NW, CHUNK).transpose(1, 0, 2)
    dst_p = jnp.concatenate([ei[1], N + lane]).reshape(
        NCHUNKS, NW, CHUNK).transpose(1, 0, 2)
    zeros_slab = jnp.zeros((ROWS_PER_TILE, H), jnp.float32)

    h0 = _h0_call(x, lin0_w, lin0_b.reshape(1, H))
    h = h0
    for l in range(L):
        beta = float(np.log(THETA / (l + 1) + 1.0))
        agg2 = _spmm(h, src_p, dst_p, zeros_slab)
        h = _layer_call(beta, agg2[:N], agg2[N_PAD:N_PAD + N], h0, conv_w[l])

    w4 = jnp.zeros((H, H, CPAD), jnp.float32).at[:, :, :C].set(
        lin1_w.reshape(H, H, C).transpose(1, 0, 2)).reshape(H, H * CPAD)
    return _final_call(h, w4, lin1_b.reshape(1, C))


# speedup vs baseline: 8.9126x; 1.0279x over previous
import functools

import numpy as np
import jax
import jax.numpy as jnp
from jax import lax
from jax.experimental import pallas as pl
from jax.experimental.pallas import tpu as pltpu
from jax.experimental.pallas import tpu_sc as plsc

N = 10000
E = 320000
D = 128
H = 64
C = 40
L = 4
ALPHA = 0.1
THETA = 0.5
MIN_NORM = 1e-15
EPS = 4e-3
MAXNORM = 1.0 - EPS

NC = 2
NS = 16
NW = NC * NS
CHUNK = 128
NCHUNKS = 80
E_PAD = NW * NCHUNKS * CHUNK
N_PAD = 12000
ROWS_PER_TILE = N_PAD // NS
CPAD = 128


def _spmm_body(h_hbm, srcp_hbm, dstp_hbm, zeros_hbm, agg_hbm,
               src_v, dst_v, rows_v, agg_sh, sem0, sem1):
    cid = lax.axis_index("c")
    sid = lax.axis_index("s")
    wid = cid * NS + sid

    pltpu.sync_copy(srcp_hbm.at[wid], src_v)
    pltpu.sync_copy(dstp_hbm.at[wid], dst_v)
    pltpu.sync_copy(zeros_hbm, agg_sh.at[pl.ds(sid * ROWS_PER_TILE, ROWS_PER_TILE)])
    plsc.subcore_barrier()

    def body(it, carry):
        j0 = it * 2
        cp0 = pltpu.async_copy(h_hbm.at[src_v.at[j0]], rows_v.at[0], sem0)
        cp1 = pltpu.async_copy(h_hbm.at[src_v.at[j0 + 1]], rows_v.at[1], sem1)
        cp0.wait()
        pltpu.sync_copy(rows_v.at[0], agg_sh.at[dst_v.at[j0]], add=True)
        cp1.wait()
        pltpu.sync_copy(rows_v.at[1], agg_sh.at[dst_v.at[j0 + 1]], add=True)
        return carry

    lax.fori_loop(0, NCHUNKS // 2, body, 0)
    plsc.subcore_barrier()
    out_base = cid * N_PAD + sid * ROWS_PER_TILE
    pltpu.sync_copy(agg_sh.at[pl.ds(sid * ROWS_PER_TILE, ROWS_PER_TILE)],
                    agg_hbm.at[pl.ds(out_base, ROWS_PER_TILE)])


_spmm = functools.partial(
    pl.kernel,
    out_type=jax.ShapeDtypeStruct((NC * N_PAD, H), jnp.float32),
    mesh=plsc.VectorSubcoreMesh(core_axis_name="c", subcore_axis_name="s"),
    scratch_types=[
        pltpu.VMEM((NCHUNKS, CHUNK), jnp.int32),
        pltpu.VMEM((NCHUNKS, CHUNK), jnp.int32),
        pltpu.VMEM((2, CHUNK, H), jnp.float32),
        pltpu.VMEM_SHARED((N_PAD, H), jnp.float32),
        pltpu.SemaphoreType.DMA,
        pltpu.SemaphoreType.DMA,
    ],
    compiler_params=pltpu.CompilerParams(use_tc_tiling_on_sc=False),
)(_spmm_body)


BN0 = 2000


def _h0_body(x_ref, w_ref, b_ref, o_ref):
    o_ref[...] = jnp.maximum(
        jnp.dot(x_ref[...], w_ref[...], preferred_element_type=jnp.float32)
        + b_ref[...], 0.0)


def _h0_call(x, w, b):
    return pl.pallas_call(
        _h0_body,
        grid=(N // BN0,),
        in_specs=[
            pl.BlockSpec((BN0, D), lambda i: (i, 0)),
            pl.BlockSpec((D, H), lambda i: (0, 0)),
            pl.BlockSpec((1, H), lambda i: (0, 0)),
        ],
        out_specs=pl.BlockSpec((BN0, H), lambda i: (i, 0)),
        out_shape=jax.ShapeDtypeStruct((N, H), jnp.float32),
    )(x, w, b)


def _layer_body(beta, a0_ref, a1_ref, h0_ref, w_ref, o_ref):
    agg = a0_ref[...] + a1_ref[...]
    hh = (1.0 - ALPHA) * agg + ALPHA * h0_ref[...]
    out = (1.0 - beta) * hh + beta * jnp.dot(
        hh, w_ref[...], preferred_element_type=jnp.float32)
    o_ref[...] = jnp.maximum(out, 0.0)


def _layer_call(beta, agg2, h0, w):
    return pl.pallas_call(
        functools.partial(_layer_body, beta),
        grid=(N // BN0,),
        in_specs=[
            pl.BlockSpec((BN0, H), lambda i: (i, 0)),
            pl.BlockSpec((BN0, H), lambda i: (i + N_PAD // BN0, 0)),
            pl.BlockSpec((BN0, H), lambda i: (i, 0)),
            pl.BlockSpec((H, H), lambda i: (0, 0)),
        ],
        out_specs=pl.BlockSpec((BN0, H), lambda i: (i, 0)),
        out_shape=jax.ShapeDtypeStruct((N, H), jnp.float32),
    )(agg2, agg2, h0, w)


BNF = 400


def _final_body(h_ref, w4_ref, b_ref, o_ref):
    h = h_ref[...]
    t = jnp.dot(h.astype(jnp.bfloat16), w4_ref[...],
                preferred_element_type=jnp.float32)
    acc = jnp.zeros((BNF, CPAD), jnp.float32)
    for i in range(H):
        acc = acc + h[:, i:i + 1] * t[:, i * CPAD:(i + 1) * CPAD]
    q = acc[:, :C]
    s1 = jnp.sum(h * h, axis=1, keepdims=True)
    nrm = jnp.maximum(s1, MIN_NORM)
    f1 = jnp.where(nrm > MAXNORM, MAXNORM / nrm, 1.0)
    pn = jnp.maximum(nrm * f1, MIN_NORM)
    pc = jnp.minimum(pn, 1.0 - 1e-7)
    art = 0.5 * (jnp.log1p(pc) - jnp.log1p(-pc))
    scale = f1 * art / pn
    y = scale * q + b_ref[...]
    un = jnp.maximum(jnp.sqrt(jnp.sum(y * y, axis=1, keepdims=True)), MIN_NORM)
    res = jnp.tanh(un) * y / un
    rn = jnp.maximum(jnp.sqrt(jnp.sum(res * res, axis=1, keepdims=True)), MIN_NORM)
    res = jnp.where(rn > MAXNORM, res / rn * MAXNORM, res)
    m = jnp.max(res, axis=1, keepdims=True)
    z = res - m
    o_ref[...] = z - jnp.log(jnp.sum(jnp.exp(z), axis=1, keepdims=True))


def _final_call(h, w4, b):
    return pl.pallas_call(
        _final_body,
        grid=(N // BNF,),
        in_specs=[
            pl.BlockSpec((BNF, H), lambda i: (i, 0)),
            pl.BlockSpec((H, H * CPAD), lambda i: (0, 0)),
            pl.BlockSpec((1, C), lambda i: (0, 0)),
        ],
        out_specs=pl.BlockSpec((BNF, C), lambda i: (i, 0)),
        out_shape=jax.ShapeDtypeStruct((N, C), jnp.float32),
    )(h, w4, b)


def kernel(x, edge_index, lin0_w, lin0_b, conv_w, lin1_w, lin1_b):
    x = x.astype(jnp.float32)
    ei = edge_index.astype(jnp.int32)
    lane = jnp.arange(E_PAD - E, dtype=jnp.int32) % CHUNK
    src_p = jnp.concatenate([ei[0], lane]).reshape(
        NCHUNKS, NW, CHUNK).transpose(1, 0, 2)
    dst_p = jnp.concatenate([ei[1], N + lane]).reshape(
        NCHUNKS, NW, CHUNK).transpose(1, 0, 2)
    zeros_slab = jnp.zeros((ROWS_PER_TILE, H), jnp.float32)

    h0 = _h0_call(x, lin0_w, lin0_b.reshape(1, H))
    h = h0
    for l in range(L):
        beta = float(np.log(THETA / (l + 1) + 1.0))
        agg2 = _spmm(h, src_p, dst_p, zeros_slab)
        h = _layer_call(beta, agg2, h0, conv_w[l])

    w4 = jnp.zeros((H, H, CPAD), jnp.float32).at[:, :, :C].set(
        lin1_w.reshape(H, H, C).transpose(1, 0, 2)).reshape(
            H, H * CPAD).astype(jnp.bfloat16)
    return _final_call(h, w4, lin1_b.reshape(1, C))


# speedup vs baseline: 10.4487x; 1.1724x over previous
import functools

import numpy as np
import jax
import jax.numpy as jnp
from jax import lax
from jax.experimental import pallas as pl
from jax.experimental.pallas import tpu as pltpu
from jax.experimental.pallas import tpu_sc as plsc

N = 10000
E = 320000
D = 128
H = 64
C = 40
L = 4
ALPHA = 0.1
THETA = 0.5
MIN_NORM = 1e-15
EPS = 4e-3
MAXNORM = 1.0 - EPS

NC = 2
NS = 16
NW = NC * NS
CHUNK = 128
NCHUNKS = 80
E_PAD = NW * NCHUNKS * CHUNK
N_PAD = 12000
ROWS_PER_TILE = N_PAD // NS
CPAD = 128


def _spmm_body(h_hbm, srcp_hbm, dstp_hbm, zeros_hbm, agg_hbm,
               src_v, dst_v, rows_v, agg_sh, sem0, sem1):
    cid = lax.axis_index("c")
    sid = lax.axis_index("s")
    wid = cid * NS + sid

    pltpu.sync_copy(srcp_hbm.at[wid], src_v)
    pltpu.sync_copy(dstp_hbm.at[wid], dst_v)
    pltpu.sync_copy(zeros_hbm, agg_sh.at[pl.ds(sid * ROWS_PER_TILE, ROWS_PER_TILE)])
    plsc.subcore_barrier()

    def body(it, carry):
        j0 = it * 2
        cp0 = pltpu.async_copy(h_hbm.at[src_v.at[j0]], rows_v.at[0], sem0)
        cp1 = pltpu.async_copy(h_hbm.at[src_v.at[j0 + 1]], rows_v.at[1], sem1)
        cp0.wait()
        pltpu.sync_copy(rows_v.at[0], agg_sh.at[dst_v.at[j0]], add=True)
        cp1.wait()
        pltpu.sync_copy(rows_v.at[1], agg_sh.at[dst_v.at[j0 + 1]], add=True)
        return carry

    lax.fori_loop(0, NCHUNKS // 2, body, 0)
    plsc.subcore_barrier()
    out_base = cid * N_PAD + sid * ROWS_PER_TILE
    pltpu.sync_copy(agg_sh.at[pl.ds(sid * ROWS_PER_TILE, ROWS_PER_TILE)],
                    agg_hbm.at[pl.ds(out_base, ROWS_PER_TILE)])


_spmm = functools.partial(
    pl.kernel,
    out_type=jax.ShapeDtypeStruct((NC * N_PAD, H), jnp.bfloat16),
    mesh=plsc.VectorSubcoreMesh(core_axis_name="c", subcore_axis_name="s"),
    scratch_types=[
        pltpu.VMEM((NCHUNKS, CHUNK), jnp.int32),
        pltpu.VMEM((NCHUNKS, CHUNK), jnp.int32),
        pltpu.VMEM((2, CHUNK, H), jnp.bfloat16),
        pltpu.VMEM_SHARED((N_PAD, H), jnp.bfloat16),
        pltpu.SemaphoreType.DMA,
        pltpu.SemaphoreType.DMA,
    ],
    compiler_params=pltpu.CompilerParams(use_tc_tiling_on_sc=False),
)(_spmm_body)


BN0 = 2000


def _h0_body(x_ref, w_ref, b_ref, o_ref, ob_ref):
    h = jnp.maximum(
        jnp.dot(x_ref[...], w_ref[...], preferred_element_type=jnp.float32)
        + b_ref[...], 0.0)
    o_ref[...] = h
    ob_ref[...] = h.astype(jnp.bfloat16)


def _h0_call(x, w, b):
    return pl.pallas_call(
        _h0_body,
        grid=(N // BN0,),
        in_specs=[
            pl.BlockSpec((BN0, D), lambda i: (i, 0)),
            pl.BlockSpec((D, H), lambda i: (0, 0)),
            pl.BlockSpec((1, H), lambda i: (0, 0)),
        ],
        out_specs=[
            pl.BlockSpec((BN0, H), lambda i: (i, 0)),
            pl.BlockSpec((BN0, H), lambda i: (i, 0)),
        ],
        out_shape=[
            jax.ShapeDtypeStruct((N, H), jnp.float32),
            jax.ShapeDtypeStruct((N, H), jnp.bfloat16),
        ],
    )(x, w, b)


def _layer_body(beta, a0_ref, a1_ref, h0_ref, w_ref, o_ref, ob_ref):
    agg = a0_ref[...].astype(jnp.float32) + a1_ref[...].astype(jnp.float32)
    hh = (1.0 - ALPHA) * agg + ALPHA * h0_ref[...]
    out = (1.0 - beta) * hh + beta * jnp.dot(
        hh, w_ref[...], preferred_element_type=jnp.float32)
    h = jnp.maximum(out, 0.0)
    o_ref[...] = h
    ob_ref[...] = h.astype(jnp.bfloat16)


def _layer_call(beta, agg2, h0, w):
    return pl.pallas_call(
        functools.partial(_layer_body, beta),
        grid=(N // BN0,),
        in_specs=[
            pl.BlockSpec((BN0, H), lambda i: (i, 0)),
            pl.BlockSpec((BN0, H), lambda i: (i + N_PAD // BN0, 0)),
            pl.BlockSpec((BN0, H), lambda i: (i, 0)),
            pl.BlockSpec((H, H), lambda i: (0, 0)),
        ],
        out_specs=[
            pl.BlockSpec((BN0, H), lambda i: (i, 0)),
            pl.BlockSpec((BN0, H), lambda i: (i, 0)),
        ],
        out_shape=[
            jax.ShapeDtypeStruct((N, H), jnp.float32),
            jax.ShapeDtypeStruct((N, H), jnp.bfloat16),
        ],
    )(agg2, agg2, h0, w)


BNF = 400


def _final_body(h_ref, w4_ref, b_ref, o_ref):
    h = h_ref[...]
    t = jnp.dot(h.astype(jnp.bfloat16), w4_ref[...],
                preferred_element_type=jnp.float32)
    acc = jnp.zeros((BNF, CPAD), jnp.float32)
    for i in range(H):
        acc = acc + h[:, i:i + 1] * t[:, i * CPAD:(i + 1) * CPAD]
    q = acc[:, :C]
    s1 = jnp.sum(h * h, axis=1, keepdims=True)
    nrm = jnp.maximum(s1, MIN_NORM)
    f1 = jnp.where(nrm > MAXNORM, MAXNORM / nrm, 1.0)
    pn = jnp.maximum(nrm * f1, MIN_NORM)
    pc = jnp.minimum(pn, 1.0 - 1e-7)
    art = 0.5 * (jnp.log1p(pc) - jnp.log1p(-pc))
    scale = f1 * art / pn
    y = scale * q + b_ref[...]
    un = jnp.maximum(jnp.sqrt(jnp.sum(y * y, axis=1, keepdims=True)), MIN_NORM)
    res = jnp.tanh(un) * y / un
    rn = jnp.maximum(jnp.sqrt(jnp.sum(res * res, axis=1, keepdims=True)), MIN_NORM)
    res = jnp.where(rn > MAXNORM, res / rn * MAXNORM, res)
    m = jnp.max(res, axis=1, keepdims=True)
    z = res - m
    o_ref[...] = z - jnp.log(jnp.sum(jnp.exp(z), axis=1, keepdims=True))


def _final_call(h, w4, b):
    return pl.pallas_call(
        _final_body,
        grid=(N // BNF,),
        in_specs=[
            pl.BlockSpec((BNF, H), lambda i: (i, 0)),
            pl.BlockSpec((H, H * CPAD), lambda i: (0, 0)),
            pl.BlockSpec((1, C), lambda i: (0, 0)),
        ],
        out_specs=pl.BlockSpec((BNF, C), lambda i: (i, 0)),
        out_shape=jax.ShapeDtypeStruct((N, C), jnp.float32),
    )(h, w4, b)


def kernel(x, edge_index, lin0_w, lin0_b, conv_w, lin1_w, lin1_b):
    x = x.astype(jnp.float32)
    ei = edge_index.astype(jnp.int32)
    lane = jnp.arange(E_PAD - E, dtype=jnp.int32) % CHUNK
    src_p = jnp.concatenate([ei[0], lane]).reshape(
        NCHUNKS, NW, CHUNK).transpose(1, 0, 2)
    dst_p = jnp.concatenate([ei[1], N + lane]).reshape(
        NCHUNKS, NW, CHUNK).transpose(1, 0, 2)
    zeros_slab = jnp.zeros((ROWS_PER_TILE, H), jnp.bfloat16)

    h0, h_bf = _h0_call(x, lin0_w, lin0_b.reshape(1, H))
    h = h0
    for l in range(L):
        beta = float(np.log(THETA / (l + 1) + 1.0))
        agg2 = _spmm(h_bf, src_p, dst_p, zeros_slab)
        h, h_bf = _layer_call(beta, agg2, h0, conv_w[l])

    w4 = jnp.zeros((H, H, CPAD), jnp.float32).at[:, :, :C].set(
        lin1_w.reshape(H, H, C).transpose(1, 0, 2)).reshape(
            H, H * CPAD).astype(jnp.bfloat16)
    return _final_call(h, w4, lin1_b.reshape(1, C))


# speedup vs baseline: 12.4457x; 1.1911x over previous
import functools

import numpy as np
import jax
import jax.numpy as jnp
from jax import lax
from jax.experimental import pallas as pl
from jax.experimental.pallas import tpu as pltpu
from jax.experimental.pallas import tpu_sc as plsc

N = 10000
E = 320000
D = 128
H = 64
C = 40
L = 4
ALPHA = 0.1
THETA = 0.5
MIN_NORM = 1e-15
EPS = 4e-3
MAXNORM = 1.0 - EPS

NC = 2
NS = 16
NW = NC * NS
CHUNK = 128
NCHUNKS = 80
E_PAD = NW * NCHUNKS * CHUNK
N_PAD = 12000
ROWS_PER_TILE = N_PAD // NS
CPAD = 128


NBUF = 4


def _spmm_body(h_hbm, srcp_hbm, dstp_hbm, zeros_hbm, agg_hbm,
               src_v, dst_v, rows_v, agg_sh, gsems, ssems):
    cid = lax.axis_index("c")
    sid = lax.axis_index("s")
    wid = cid * NS + sid

    pltpu.sync_copy(srcp_hbm.at[wid], src_v)
    pltpu.sync_copy(dstp_hbm.at[wid], dst_v)
    pltpu.sync_copy(zeros_hbm, agg_sh.at[pl.ds(sid * ROWS_PER_TILE, ROWS_PER_TILE)])
    plsc.subcore_barrier()

    def gather(j, b):
        return pltpu.make_async_copy(h_hbm.at[src_v.at[j]], rows_v.at[b], gsems[b])

    def scatter(j, b):
        return pltpu.make_async_copy(rows_v.at[b], agg_sh.at[dst_v.at[j]], ssems[b])

    for b in range(NBUF):
        pltpu.async_copy(h_hbm.at[src_v.at[b]], rows_v.at[b], gsems[b])
    for b in range(NBUF):
        gather(b, b).wait()
        pltpu.async_copy(rows_v.at[b], agg_sh.at[dst_v.at[b]], ssems[b], add=True)

    def body(it, carry):
        j0 = it * NBUF
        for b in range(NBUF):
            j = j0 + b
            scatter(j - NBUF, b).wait()
            pltpu.async_copy(h_hbm.at[src_v.at[j]], rows_v.at[b], gsems[b])
        for b in range(NBUF):
            j = j0 + b
            gather(j, b).wait()
            pltpu.async_copy(rows_v.at[b], agg_sh.at[dst_v.at[j]], ssems[b], add=True)
        return carry

    lax.fori_loop(1, NCHUNKS // NBUF, body, 0)
    for b in range(NBUF):
        scatter(NCHUNKS - NBUF + b, b).wait()
    plsc.subcore_barrier()
    out_base = cid * N_PAD + sid * ROWS_PER_TILE
    pltpu.sync_copy(agg_sh.at[pl.ds(sid * ROWS_PER_TILE, ROWS_PER_TILE)],
                    agg_hbm.at[pl.ds(out_base, ROWS_PER_TILE)])


_spmm = functools.partial(
    pl.kernel,
    out_type=jax.ShapeDtypeStruct((NC * N_PAD, H), jnp.bfloat16),
    mesh=plsc.VectorSubcoreMesh(core_axis_name="c", subcore_axis_name="s"),
    scratch_types=[
        pltpu.VMEM((NCHUNKS, CHUNK), jnp.int32),
        pltpu.VMEM((NCHUNKS, CHUNK), jnp.int32),
        pltpu.VMEM((NBUF, CHUNK, H), jnp.bfloat16),
        pltpu.VMEM_SHARED((N_PAD, H), jnp.bfloat16),
        [pltpu.SemaphoreType.DMA] * NBUF,
        [pltpu.SemaphoreType.DMA] * NBUF,
    ],
    compiler_params=pltpu.CompilerParams(use_tc_tiling_on_sc=False),
)(_spmm_body)


BN0 = 2000


def _h0_body(x_ref, w_ref, b_ref, o_ref, ob_ref):
    h = jnp.maximum(
        jnp.dot(x_ref[...], w_ref[...], preferred_element_type=jnp.float32)
        + b_ref[...], 0.0)
    o_ref[...] = h
    ob_ref[...] = h.astype(jnp.bfloat16)


def _h0_call(x, w, b):
    return pl.pallas_call(
        _h0_body,
        grid=(N // BN0,),
        in_specs=[
            pl.BlockSpec((BN0, D), lambda i: (i, 0)),
            pl.BlockSpec((D, H), lambda i: (0, 0)),
            pl.BlockSpec((1, H), lambda i: (0, 0)),
        ],
        out_specs=[
            pl.BlockSpec((BN0, H), lambda i: (i, 0)),
            pl.BlockSpec((BN0, H), lambda i: (i, 0)),
        ],
        out_shape=[
            jax.ShapeDtypeStruct((N, H), jnp.float32),
            jax.ShapeDtypeStruct((N, H), jnp.bfloat16),
        ],
    )(x, w, b)


def _layer_body(beta, a0_ref, a1_ref, h0_ref, w_ref, o_ref, ob_ref):
    agg = a0_ref[...].astype(jnp.float32) + a1_ref[...].astype(jnp.float32)
    hh = (1.0 - ALPHA) * agg + ALPHA * h0_ref[...]
    out = (1.0 - beta) * hh + beta * jnp.dot(
        hh, w_ref[...], preferred_element_type=jnp.float32)
    h = jnp.maximum(out, 0.0)
    o_ref[...] = h
    ob_ref[...] = h.astype(jnp.bfloat16)


def _layer_call(beta, agg2, h0, w):
    return pl.pallas_call(
        functools.partial(_layer_body, beta),
        grid=(N // BN0,),
        in_specs=[
            pl.BlockSpec((BN0, H), lambda i: (i, 0)),
            pl.BlockSpec((BN0, H), lambda i: (i + N_PAD // BN0, 0)),
            pl.BlockSpec((BN0, H), lambda i: (i, 0)),
            pl.BlockSpec((H, H), lambda i: (0, 0)),
        ],
        out_specs=[
            pl.BlockSpec((BN0, H), lambda i: (i, 0)),
            pl.BlockSpec((BN0, H), lambda i: (i, 0)),
        ],
        out_shape=[
            jax.ShapeDtypeStruct((N, H), jnp.float32),
            jax.ShapeDtypeStruct((N, H), jnp.bfloat16),
        ],
    )(agg2, agg2, h0, w)


BNF = 400


def _final_body(h_ref, w4_ref, b_ref, o_ref):
    h = h_ref[...]
    t = jnp.dot(h.astype(jnp.bfloat16), w4_ref[...],
                preferred_element_type=jnp.float32)
    acc = jnp.zeros((BNF, CPAD), jnp.float32)
    for i in range(H):
        acc = acc + h[:, i:i + 1] * t[:, i * CPAD:(i + 1) * CPAD]
    q = acc[:, :C]
    s1 = jnp.sum(h * h, axis=1, keepdims=True)
    nrm = jnp.maximum(s1, MIN_NORM)
    f1 = jnp.where(nrm > MAXNORM, MAXNORM / nrm, 1.0)
    pn = jnp.maximum(nrm * f1, MIN_NORM)
    pc = jnp.minimum(pn, 1.0 - 1e-7)
    art = 0.5 * (jnp.log1p(pc) - jnp.log1p(-pc))
    scale = f1 * art / pn
    y = scale * q + b_ref[...]
    un = jnp.maximum(jnp.sqrt(jnp.sum(y * y, axis=1, keepdims=True)), MIN_NORM)
    res = jnp.tanh(un) * y / un
    rn = jnp.maximum(jnp.sqrt(jnp.sum(res * res, axis=1, keepdims=True)), MIN_NORM)
    res = jnp.where(rn > MAXNORM, res / rn * MAXNORM, res)
    m = jnp.max(res, axis=1, keepdims=True)
    z = res - m
    o_ref[...] = z - jnp.log(jnp.sum(jnp.exp(z), axis=1, keepdims=True))


def _final_call(h, w4, b):
    return pl.pallas_call(
        _final_body,
        grid=(N // BNF,),
        in_specs=[
            pl.BlockSpec((BNF, H), lambda i: (i, 0)),
            pl.BlockSpec((H, H * CPAD), lambda i: (0, 0)),
            pl.BlockSpec((1, C), lambda i: (0, 0)),
        ],
        out_specs=pl.BlockSpec((BNF, C), lambda i: (i, 0)),
        out_shape=jax.ShapeDtypeStruct((N, C), jnp.float32),
    )(h, w4, b)


def kernel(x, edge_index, lin0_w, lin0_b, conv_w, lin1_w, lin1_b):
    x = x.astype(jnp.float32)
    ei = edge_index.astype(jnp.int32)
    lane = jnp.arange(E_PAD - E, dtype=jnp.int32) % CHUNK
    src_p = jnp.concatenate([ei[0], lane]).reshape(
        NCHUNKS, NW, CHUNK).transpose(1, 0, 2)
    dst_p = jnp.concatenate([ei[1], N + lane]).reshape(
        NCHUNKS, NW, CHUNK).transpose(1, 0, 2)
    zeros_slab = jnp.zeros((ROWS_PER_TILE, H), jnp.bfloat16)

    h0, h_bf = _h0_call(x, lin0_w, lin0_b.reshape(1, H))
    h = h0
    for l in range(L):
        beta = float(np.log(THETA / (l + 1) + 1.0))
        agg2 = _spmm(h_bf, src_p, dst_p, zeros_slab)
        h, h_bf = _layer_call(beta, agg2, h0, conv_w[l])

    w4 = jnp.zeros((H, H, CPAD), jnp.float32).at[:, :, :C].set(
        lin1_w.reshape(H, H, C).transpose(1, 0, 2)).reshape(
            H, H * CPAD).astype(jnp.bfloat16)
    return _final_call(h, w4, lin1_b.reshape(1, C))


# speedup vs baseline: 12.9203x; 1.0381x over previous
import functools

import numpy as np
import jax
import jax.numpy as jnp
from jax import lax
from jax.experimental import pallas as pl
from jax.experimental.pallas import tpu as pltpu
from jax.experimental.pallas import tpu_sc as plsc

N = 10000
E = 320000
D = 128
H = 64
C = 40
L = 4
ALPHA = 0.1
THETA = 0.5
MIN_NORM = 1e-15
EPS = 4e-3
MAXNORM = 1.0 - EPS

NC = 2
NS = 16
NW = NC * NS
CHUNK = 128
NCHUNKS = 80
E_PAD = NW * NCHUNKS * CHUNK
N_PAD = 12000
ROWS_PER_TILE = N_PAD // NS
CPAD = 128


NBUF = 8


def _spmm_body(h_hbm, srcp_hbm, dstp_hbm, zeros_hbm, agg_hbm,
               src_v, dst_v, rows_v, agg_sh, gsems, ssems):
    cid = lax.axis_index("c")
    sid = lax.axis_index("s")
    wid = cid * NS + sid

    pltpu.sync_copy(srcp_hbm.at[wid], src_v)
    pltpu.sync_copy(dstp_hbm.at[wid], dst_v)
    pltpu.sync_copy(zeros_hbm, agg_sh.at[pl.ds(sid * ROWS_PER_TILE, ROWS_PER_TILE)])
    plsc.subcore_barrier()

    def gather(j, b):
        return pltpu.make_async_copy(h_hbm.at[src_v.at[j]], rows_v.at[b], gsems[b])

    def scatter(j, b):
        return pltpu.make_async_copy(rows_v.at[b], agg_sh.at[dst_v.at[j]], ssems[b])

    for b in range(NBUF):
        pltpu.async_copy(h_hbm.at[src_v.at[b]], rows_v.at[b], gsems[b])
    for b in range(NBUF):
        gather(b, b).wait()
        pltpu.async_copy(rows_v.at[b], agg_sh.at[dst_v.at[b]], ssems[b], add=True)

    def body(it, carry):
        j0 = it * NBUF
        for b in range(NBUF):
            j = j0 + b
            scatter(j - NBUF, b).wait()
            pltpu.async_copy(h_hbm.at[src_v.at[j]], rows_v.at[b], gsems[b])
        for b in range(NBUF):
            j = j0 + b
            gather(j, b).wait()
            pltpu.async_copy(rows_v.at[b], agg_sh.at[dst_v.at[j]], ssems[b], add=True)
        return carry

    lax.fori_loop(1, NCHUNKS // NBUF, body, 0)
    for b in range(NBUF):
        scatter(NCHUNKS - NBUF + b, b).wait()
    plsc.subcore_barrier()
    out_base = cid * N_PAD + sid * ROWS_PER_TILE
    pltpu.sync_copy(agg_sh.at[pl.ds(sid * ROWS_PER_TILE, ROWS_PER_TILE)],
                    agg_hbm.at[pl.ds(out_base, ROWS_PER_TILE)])


_spmm = functools.partial(
    pl.kernel,
    out_type=jax.ShapeDtypeStruct((NC * N_PAD, H), jnp.bfloat16),
    mesh=plsc.VectorSubcoreMesh(core_axis_name="c", subcore_axis_name="s"),
    scratch_types=[
        pltpu.VMEM((NCHUNKS, CHUNK), jnp.int32),
        pltpu.VMEM((NCHUNKS, CHUNK), jnp.int32),
        pltpu.VMEM((NBUF, CHUNK, H), jnp.bfloat16),
        pltpu.VMEM_SHARED((N_PAD, H), jnp.bfloat16),
        [pltpu.SemaphoreType.DMA] * NBUF,
        [pltpu.SemaphoreType.DMA] * NBUF,
    ],
    compiler_params=pltpu.CompilerParams(use_tc_tiling_on_sc=False),
)(_spmm_body)


BN0 = 2000


def _h0_body(x_ref, w_ref, b_ref, o_ref, ob_ref):
    h = jnp.maximum(
        jnp.dot(x_ref[...], w_ref[...], preferred_element_type=jnp.float32)
        + b_ref[...], 0.0)
    o_ref[...] = h
    ob_ref[...] = h.astype(jnp.bfloat16)


def _h0_call(x, w, b):
    return pl.pallas_call(
        _h0_body,
        grid=(N // BN0,),
        in_specs=[
            pl.BlockSpec((BN0, D), lambda i: (i, 0)),
            pl.BlockSpec((D, H), lambda i: (0, 0)),
            pl.BlockSpec((1, H), lambda i: (0, 0)),
        ],
        out_specs=[
            pl.BlockSpec((BN0, H), lambda i: (i, 0)),
            pl.BlockSpec((BN0, H), lambda i: (i, 0)),
        ],
        out_shape=[
            jax.ShapeDtypeStruct((N, H), jnp.float32),
            jax.ShapeDtypeStruct((N, H), jnp.bfloat16),
        ],
    )(x, w, b)


def _layer_body(beta, a0_ref, a1_ref, h0_ref, w_ref, o_ref, ob_ref):
    agg = a0_ref[...].astype(jnp.float32) + a1_ref[...].astype(jnp.float32)
    hh = (1.0 - ALPHA) * agg + ALPHA * h0_ref[...]
    out = (1.0 - beta) * hh + beta * jnp.dot(
        hh, w_ref[...], preferred_element_type=jnp.float32)
    h = jnp.maximum(out, 0.0)
    o_ref[...] = h
    ob_ref[...] = h.astype(jnp.bfloat16)


def _layer_call(beta, agg2, h0, w):
    return pl.pallas_call(
        functools.partial(_layer_body, beta),
        grid=(N // BN0,),
        in_specs=[
            pl.BlockSpec((BN0, H), lambda i: (i, 0)),
            pl.BlockSpec((BN0, H), lambda i: (i + N_PAD // BN0, 0)),
            pl.BlockSpec((BN0, H), lambda i: (i, 0)),
            pl.BlockSpec((H, H), lambda i: (0, 0)),
        ],
        out_specs=[
            pl.BlockSpec((BN0, H), lambda i: (i, 0)),
            pl.BlockSpec((BN0, H), lambda i: (i, 0)),
        ],
        out_shape=[
            jax.ShapeDtypeStruct((N, H), jnp.float32),
            jax.ShapeDtypeStruct((N, H), jnp.bfloat16),
        ],
    )(agg2, agg2, h0, w)


BNF = 400


def _final_body(h_ref, w4_ref, b_ref, o_ref):
    h = h_ref[...]
    t = jnp.dot(h.astype(jnp.bfloat16), w4_ref[...],
                preferred_element_type=jnp.float32)
    acc = jnp.zeros((BNF, CPAD), jnp.float32)
    for i in range(H):
        acc = acc + h[:, i:i + 1] * t[:, i * CPAD:(i + 1) * CPAD]
    q = acc[:, :C]
    s1 = jnp.sum(h * h, axis=1, keepdims=True)
    nrm = jnp.maximum(s1, MIN_NORM)
    f1 = jnp.where(nrm > MAXNORM, MAXNORM / nrm, 1.0)
    pn = jnp.maximum(nrm * f1, MIN_NORM)
    pc = jnp.minimum(pn, 1.0 - 1e-7)
    art = 0.5 * (jnp.log1p(pc) - jnp.log1p(-pc))
    scale = f1 * art / pn
    y = scale * q + b_ref[...]
    un = jnp.maximum(jnp.sqrt(jnp.sum(y * y, axis=1, keepdims=True)), MIN_NORM)
    res = jnp.tanh(un) * y / un
    rn = jnp.maximum(jnp.sqrt(jnp.sum(res * res, axis=1, keepdims=True)), MIN_NORM)
    res = jnp.where(rn > MAXNORM, res / rn * MAXNORM, res)
    m = jnp.max(res, axis=1, keepdims=True)
    z = res - m
    o_ref[...] = z - jnp.log(jnp.sum(jnp.exp(z), axis=1, keepdims=True))


def _final_call(h, w4, b):
    return pl.pallas_call(
        _final_body,
        grid=(N // BNF,),
        in_specs=[
            pl.BlockSpec((BNF, H), lambda i: (i, 0)),
            pl.BlockSpec((H, H * CPAD), lambda i: (0, 0)),
            pl.BlockSpec((1, C), lambda i: (0, 0)),
        ],
        out_specs=pl.BlockSpec((BNF, C), lambda i: (i, 0)),
        out_shape=jax.ShapeDtypeStruct((N, C), jnp.float32),
    )(h, w4, b)


def kernel(x, edge_index, lin0_w, lin0_b, conv_w, lin1_w, lin1_b):
    x = x.astype(jnp.float32)
    ei = edge_index.astype(jnp.int32)
    lane = jnp.arange(E_PAD - E, dtype=jnp.int32) % CHUNK
    src_p = jnp.concatenate([ei[0], lane]).reshape(
        NCHUNKS, NW, CHUNK).transpose(1, 0, 2)
    dst_p = jnp.concatenate([ei[1], N + lane]).reshape(
        NCHUNKS, NW, CHUNK).transpose(1, 0, 2)
    zeros_slab = jnp.zeros((ROWS_PER_TILE, H), jnp.bfloat16)

    h0, h_bf = _h0_call(x, lin0_w, lin0_b.reshape(1, H))
    h = h0
    for l in range(L):
        beta = float(np.log(THETA / (l + 1) + 1.0))
        agg2 = _spmm(h_bf, src_p, dst_p, zeros_slab)
        h, h_bf = _layer_call(beta, agg2, h0, conv_w[l])

    w4 = jnp.zeros((H, H, CPAD), jnp.float32).at[:, :, :C].set(
        lin1_w.reshape(H, H, C).transpose(1, 0, 2)).reshape(
            H, H * CPAD).astype(jnp.bfloat16)
    return _final_call(h, w4, lin1_b.reshape(1, C))


# speedup vs baseline: 15.6214x; 1.2091x over previous
import functools

import numpy as np
import jax
import jax.numpy as jnp
from jax import lax
from jax.experimental import pallas as pl
from jax.experimental.pallas import tpu as pltpu
from jax.experimental.pallas import tpu_sc as plsc

N = 10000
E = 320000
D = 128
H = 64
C = 40
L = 4
ALPHA = 0.1
THETA = 0.5
MIN_NORM = 1e-15
EPS = 4e-3
MAXNORM = 1.0 - EPS

NC = 2
NS = 16
NW = NC * NS
CHUNK = 128
NCHUNKS = 80
E_PAD = NW * NCHUNKS * CHUNK
N_PAD = 12000
ROWS_PER_TILE = N_PAD // NS
CPAD = 128


NBUF = 8


def _spmm_body(h_hbm, srcp_hbm, dstp_hbm, zeros_hbm, agg_hbm,
               src_v, dst_v, rows_v, agg_sh, gsems, ssems):
    cid = lax.axis_index("c")
    sid = lax.axis_index("s")
    wid = cid * NS + sid

    pltpu.sync_copy(srcp_hbm.at[wid], src_v)
    pltpu.sync_copy(dstp_hbm.at[wid], dst_v)
    pltpu.sync_copy(zeros_hbm, agg_sh.at[pl.ds(sid * ROWS_PER_TILE, ROWS_PER_TILE)])
    plsc.subcore_barrier()

    def gather(j, b):
        return pltpu.make_async_copy(h_hbm.at[src_v.at[j]], rows_v.at[b], gsems[b])

    def scatter(j, b):
        return pltpu.make_async_copy(rows_v.at[b], agg_sh.at[dst_v.at[j]], ssems[b])

    for b in range(NBUF):
        pltpu.async_copy(h_hbm.at[src_v.at[b]], rows_v.at[b], gsems[b])
    for b in range(NBUF):
        gather(b, b).wait()
        pltpu.async_copy(rows_v.at[b], agg_sh.at[dst_v.at[b]], ssems[b], add=True)

    def body(it, carry):
        j0 = it * NBUF
        for b in range(NBUF):
            j = j0 + b
            scatter(j - NBUF, b).wait()
            pltpu.async_copy(h_hbm.at[src_v.at[j]], rows_v.at[b], gsems[b])
        for b in range(NBUF):
            j = j0 + b
            gather(j, b).wait()
            pltpu.async_copy(rows_v.at[b], agg_sh.at[dst_v.at[j]], ssems[b], add=True)
        return carry

    lax.fori_loop(1, NCHUNKS // NBUF, body, 0)
    for b in range(NBUF):
        scatter(NCHUNKS - NBUF + b, b).wait()
    plsc.subcore_barrier()
    out_base = cid * N_PAD + sid * ROWS_PER_TILE
    pltpu.sync_copy(agg_sh.at[pl.ds(sid * ROWS_PER_TILE, ROWS_PER_TILE)],
                    agg_hbm.at[pl.ds(out_base, ROWS_PER_TILE)])


_spmm = functools.partial(
    pl.kernel,
    out_type=jax.ShapeDtypeStruct((NC * N_PAD, H), jnp.bfloat16),
    mesh=plsc.VectorSubcoreMesh(core_axis_name="c", subcore_axis_name="s"),
    scratch_types=[
        pltpu.VMEM((NCHUNKS, CHUNK), jnp.int32),
        pltpu.VMEM((NCHUNKS, CHUNK), jnp.int32),
        pltpu.VMEM((NBUF, CHUNK, H), jnp.bfloat16),
        pltpu.VMEM_SHARED((N_PAD, H), jnp.bfloat16),
        [pltpu.SemaphoreType.DMA] * NBUF,
        [pltpu.SemaphoreType.DMA] * NBUF,
    ],
    compiler_params=pltpu.CompilerParams(use_tc_tiling_on_sc=False),
)(_spmm_body)


BN0 = 2000


def _h0_body(x_ref, w_ref, b_ref, o_ref, ob_ref):
    h = jnp.maximum(
        jnp.dot(x_ref[...], w_ref[...], preferred_element_type=jnp.float32)
        + b_ref[...], 0.0)
    o_ref[...] = h
    ob_ref[...] = h.astype(jnp.bfloat16)


def _h0_call(x, w, b):
    return pl.pallas_call(
        _h0_body,
        grid=(N // BN0,),
        in_specs=[
            pl.BlockSpec((BN0, D), lambda i: (i, 0)),
            pl.BlockSpec((D, H), lambda i: (0, 0)),
            pl.BlockSpec((1, H), lambda i: (0, 0)),
        ],
        out_specs=[
            pl.BlockSpec((BN0, H), lambda i: (i, 0)),
            pl.BlockSpec((BN0, H), lambda i: (i, 0)),
        ],
        out_shape=[
            jax.ShapeDtypeStruct((N, H), jnp.float32),
            jax.ShapeDtypeStruct((N, H), jnp.bfloat16),
        ],
    )(x, w, b)


def _layer_body(beta, a0_ref, a1_ref, h0_ref, w_ref, o_ref, ob_ref):
    agg = a0_ref[...].astype(jnp.float32) + a1_ref[...].astype(jnp.float32)
    hh = (1.0 - ALPHA) * agg + ALPHA * h0_ref[...]
    out = (1.0 - beta) * hh + beta * jnp.dot(
        hh, w_ref[...], preferred_element_type=jnp.float32)
    h = jnp.maximum(out, 0.0)
    o_ref[...] = h
    ob_ref[...] = h.astype(jnp.bfloat16)


def _layer_call(beta, agg2, h0, w):
    return pl.pallas_call(
        functools.partial(_layer_body, beta),
        grid=(N // BN0,),
        in_specs=[
            pl.BlockSpec((BN0, H), lambda i: (i, 0)),
            pl.BlockSpec((BN0, H), lambda i: (i + N_PAD // BN0, 0)),
            pl.BlockSpec((BN0, H), lambda i: (i, 0)),
            pl.BlockSpec((H, H), lambda i: (0, 0)),
        ],
        out_specs=[
            pl.BlockSpec((BN0, H), lambda i: (i, 0)),
            pl.BlockSpec((BN0, H), lambda i: (i, 0)),
        ],
        out_shape=[
            jax.ShapeDtypeStruct((N, H), jnp.float32),
            jax.ShapeDtypeStruct((N, H), jnp.bfloat16),
        ],
    )(agg2, agg2, h0, w)


BNF = 400


def _final_body(h_ref, w4t_ref, b_ref, o_ref):
    h = h_ref[...]
    ht = jnp.transpose(h)
    tt = jnp.dot(w4t_ref[...], ht.astype(jnp.bfloat16),
                 preferred_element_type=jnp.float32)
    acc = jnp.zeros((C, BNF), jnp.float32)
    for i in range(H):
        acc = acc + ht[i:i + 1, :] * tt[i * C:(i + 1) * C, :]
    s1 = jnp.sum(ht * ht, axis=0, keepdims=True)
    nrm = jnp.maximum(s1, MIN_NORM)
    f1 = jnp.where(nrm > MAXNORM, MAXNORM / nrm, 1.0)
    pn = jnp.maximum(nrm * f1, MIN_NORM)
    pc = jnp.minimum(pn, 1.0 - 1e-7)
    art = 0.5 * (jnp.log1p(pc) - jnp.log1p(-pc))
    scale = f1 * art / pn
    y = scale * acc + jnp.transpose(b_ref[...])
    un = jnp.maximum(jnp.sqrt(jnp.sum(y * y, axis=0, keepdims=True)), MIN_NORM)
    res = jnp.tanh(un) * y / un
    rn = jnp.maximum(jnp.sqrt(jnp.sum(res * res, axis=0, keepdims=True)), MIN_NORM)
    res = jnp.where(rn > MAXNORM, res / rn * MAXNORM, res)
    m = jnp.max(res, axis=0, keepdims=True)
    z = res - m
    out_t = z - jnp.log(jnp.sum(jnp.exp(z), axis=0, keepdims=True))
    o_ref[...] = jnp.transpose(out_t)


def _final_call(h, w4t, b):
    return pl.pallas_call(
        _final_body,
        grid=(N // BNF,),
        in_specs=[
            pl.BlockSpec((BNF, H), lambda i: (i, 0)),
            pl.BlockSpec((H * C, H), lambda i: (0, 0)),
            pl.BlockSpec((1, C), lambda i: (0, 0)),
        ],
        out_specs=pl.BlockSpec((BNF, C), lambda i: (i, 0)),
        out_shape=jax.ShapeDtypeStruct((N, C), jnp.float32),
    )(h, w4t, b)


def kernel(x, edge_index, lin0_w, lin0_b, conv_w, lin1_w, lin1_b):
    x = x.astype(jnp.float32)
    ei = edge_index.astype(jnp.int32)
    lane = jnp.arange(E_PAD - E, dtype=jnp.int32) % CHUNK
    src_p = jnp.concatenate([ei[0], lane]).reshape(
        NCHUNKS, NW, CHUNK).transpose(1, 0, 2)
    dst_p = jnp.concatenate([ei[1], N + lane]).reshape(
        NCHUNKS, NW, CHUNK).transpose(1, 0, 2)
    zeros_slab = jnp.zeros((ROWS_PER_TILE, H), jnp.bfloat16)

    h0, h_bf = _h0_call(x, lin0_w, lin0_b.reshape(1, H))
    h = h0
    for l in range(L):
        beta = float(np.log(THETA / (l + 1) + 1.0))
        agg2 = _spmm(h_bf, src_p, dst_p, zeros_slab)
        h, h_bf = _layer_call(beta, agg2, h0, conv_w[l])

    w4t = lin1_w.reshape(H, H, C).transpose(0, 2, 1).reshape(
        H * C, H).astype(jnp.bfloat16)
    return _final_call(h, w4t, lin1_b.reshape(1, C))


# speedup vs baseline: 15.8597x; 1.0153x over previous
import functools

import numpy as np
import jax
import jax.numpy as jnp
from jax import lax
from jax.experimental import pallas as pl
from jax.experimental.pallas import tpu as pltpu
from jax.experimental.pallas import tpu_sc as plsc

N = 10000
E = 320000
D = 128
H = 64
C = 40
L = 4
ALPHA = 0.1
THETA = 0.5
MIN_NORM = 1e-15
EPS = 4e-3
MAXNORM = 1.0 - EPS

NC = 2
NS = 16
NW = NC * NS
CHUNK = 128
NCHUNKS = 80
E_PAD = NW * NCHUNKS * CHUNK
N_PAD = 12000
ROWS_PER_TILE = N_PAD // NS
CPAD = 128


NBUF = 4


def _spmm_body(h_hbm, srcp_hbm, dstp_hbm, zeros_hbm, agg_hbm,
               src_v, dst_v, rows_v, agg_sh, gsems, ssems):
    cid = lax.axis_index("c")
    sid = lax.axis_index("s")
    wid = cid * NS + sid

    pltpu.sync_copy(srcp_hbm.at[wid], src_v)
    pltpu.sync_copy(dstp_hbm.at[wid], dst_v)
    pltpu.sync_copy(zeros_hbm, agg_sh.at[pl.ds(sid * ROWS_PER_TILE, ROWS_PER_TILE)])
    plsc.subcore_barrier()

    def gather(j, b):
        return pltpu.make_async_copy(h_hbm.at[src_v.at[j]], rows_v.at[b], gsems[b])

    def scatter(j, b):
        return pltpu.make_async_copy(rows_v.at[b], agg_sh.at[dst_v.at[j]], ssems[b])

    for b in range(NBUF):
        pltpu.async_copy(h_hbm.at[src_v.at[b]], rows_v.at[b], gsems[b])
    for b in range(NBUF):
        gather(b, b).wait()
        pltpu.async_copy(rows_v.at[b], agg_sh.at[dst_v.at[b]], ssems[b], add=True)

    def body(it, carry):
        j0 = it * NBUF
        for b in range(NBUF):
            j = j0 + b
            scatter(j - NBUF, b).wait()
            pltpu.async_copy(h_hbm.at[src_v.at[j]], rows_v.at[b], gsems[b])
        for b in range(NBUF):
            j = j0 + b
            gather(j, b).wait()
            pltpu.async_copy(rows_v.at[b], agg_sh.at[dst_v.at[j]], ssems[b], add=True)
        return carry

    lax.fori_loop(1, NCHUNKS // NBUF, body, 0)
    for b in range(NBUF):
        scatter(NCHUNKS - NBUF + b, b).wait()
    plsc.subcore_barrier()
    out_base = cid * N_PAD + sid * ROWS_PER_TILE
    pltpu.sync_copy(agg_sh.at[pl.ds(sid * ROWS_PER_TILE, ROWS_PER_TILE)],
                    agg_hbm.at[pl.ds(out_base, ROWS_PER_TILE)])


_spmm = functools.partial(
    pl.kernel,
    out_type=jax.ShapeDtypeStruct((NC * N_PAD, H), jnp.bfloat16),
    mesh=plsc.VectorSubcoreMesh(core_axis_name="c", subcore_axis_name="s"),
    scratch_types=[
        pltpu.VMEM((NCHUNKS, CHUNK), jnp.int32),
        pltpu.VMEM((NCHUNKS, CHUNK), jnp.int32),
        pltpu.VMEM((NBUF, CHUNK, H), jnp.bfloat16),
        pltpu.VMEM_SHARED((N_PAD, H), jnp.bfloat16),
        [pltpu.SemaphoreType.DMA] * NBUF,
        [pltpu.SemaphoreType.DMA] * NBUF,
    ],
    compiler_params=pltpu.CompilerParams(use_tc_tiling_on_sc=False),
)(_spmm_body)


BN0 = 2000


def _h0_body(x_ref, w_ref, b_ref, o_ref, ob_ref):
    h = jnp.maximum(
        jnp.dot(x_ref[...], w_ref[...], preferred_element_type=jnp.float32)
        + b_ref[...], 0.0)
    o_ref[...] = h
    ob_ref[...] = h.astype(jnp.bfloat16)


def _h0_call(x, w, b):
    return pl.pallas_call(
        _h0_body,
        grid=(N // BN0,),
        in_specs=[
            pl.BlockSpec((BN0, D), lambda i: (i, 0)),
            pl.BlockSpec((D, H), lambda i: (0, 0)),
            pl.BlockSpec((1, H), lambda i: (0, 0)),
        ],
        out_specs=[
            pl.BlockSpec((BN0, H), lambda i: (i, 0)),
            pl.BlockSpec((BN0, H), lambda i: (i, 0)),
        ],
        out_shape=[
            jax.ShapeDtypeStruct((N, H), jnp.float32),
            jax.ShapeDtypeStruct((N, H), jnp.bfloat16),
        ],
    )(x, w, b)


def _layer_body(beta, want_f32, a0_ref, a1_ref, h0_ref, w_ref, *o_refs):
    agg = a0_ref[...].astype(jnp.float32) + a1_ref[...].astype(jnp.float32)
    hh = (1.0 - ALPHA) * agg + ALPHA * h0_ref[...]
    out = (1.0 - beta) * hh + beta * jnp.dot(
        hh, w_ref[...], preferred_element_type=jnp.float32)
    h = jnp.maximum(out, 0.0)
    if want_f32:
        o_refs[0][...] = h
        o_refs[1][...] = h.astype(jnp.bfloat16)
    else:
        o_refs[0][...] = h.astype(jnp.bfloat16)


def _layer_call(beta, agg2, h0, w, want_f32):
    nout = 2 if want_f32 else 1
    shapes = ([jax.ShapeDtypeStruct((N, H), jnp.float32)] if want_f32 else []) + [
        jax.ShapeDtypeStruct((N, H), jnp.bfloat16)]
    out = pl.pallas_call(
        functools.partial(_layer_body, beta, want_f32),
        grid=(N // BN0,),
        in_specs=[
            pl.BlockSpec((BN0, H), lambda i: (i, 0)),
            pl.BlockSpec((BN0, H), lambda i: (i + N_PAD // BN0, 0)),
            pl.BlockSpec((BN0, H), lambda i: (i, 0)),
            pl.BlockSpec((H, H), lambda i: (0, 0)),
        ],
        out_specs=[pl.BlockSpec((BN0, H), lambda i: (i, 0))] * nout,
        out_shape=shapes,
    )(agg2, agg2, h0, w)
    return out if want_f32 else (None, out[0])


BNF = 1000


def _final_body(h_ref, w4t_ref, b_ref, o_ref):
    h = h_ref[...]
    ht = jnp.transpose(h)
    tt = jnp.dot(w4t_ref[...], ht.astype(jnp.bfloat16),
                 preferred_element_type=jnp.float32)
    acc = jnp.zeros((C, BNF), jnp.float32)
    for i in range(H):
        acc = acc + ht[i:i + 1, :] * tt[i * C:(i + 1) * C, :]
    s1 = jnp.sum(ht * ht, axis=0, keepdims=True)
    nrm = jnp.maximum(s1, MIN_NORM)
    f1 = jnp.where(nrm > MAXNORM, MAXNORM / nrm, 1.0)
    pn = jnp.maximum(nrm * f1, MIN_NORM)
    pc = jnp.minimum(pn, 1.0 - 1e-7)
    art = 0.5 * (jnp.log1p(pc) - jnp.log1p(-pc))
    scale = f1 * art / pn
    y = scale * acc + jnp.transpose(b_ref[...])
    un = jnp.maximum(jnp.sqrt(jnp.sum(y * y, axis=0, keepdims=True)), MIN_NORM)
    res = jnp.tanh(un) * y / un
    rn = jnp.maximum(jnp.sqrt(jnp.sum(res * res, axis=0, keepdims=True)), MIN_NORM)
    res = jnp.where(rn > MAXNORM, res / rn * MAXNORM, res)
    m = jnp.max(res, axis=0, keepdims=True)
    z = res - m
    out_t = z - jnp.log(jnp.sum(jnp.exp(z), axis=0, keepdims=True))
    o_ref[...] = jnp.transpose(out_t)


def _final_call(h, w4t, b):
    return pl.pallas_call(
        _final_body,
        grid=(N // BNF,),
        in_specs=[
            pl.BlockSpec((BNF, H), lambda i: (i, 0)),
            pl.BlockSpec((H * C, H), lambda i: (0, 0)),
            pl.BlockSpec((1, C), lambda i: (0, 0)),
        ],
        out_specs=pl.BlockSpec((BNF, C), lambda i: (i, 0)),
        out_shape=jax.ShapeDtypeStruct((N, C), jnp.float32),
    )(h, w4t, b)


def kernel(x, edge_index, lin0_w, lin0_b, conv_w, lin1_w, lin1_b):
    x = x.astype(jnp.float32)
    ei = edge_index.astype(jnp.int32)
    lane = jnp.arange(E_PAD - E, dtype=jnp.int32) % CHUNK
    src_p = jnp.concatenate([ei[0], lane]).reshape(NW, NCHUNKS, CHUNK)
    dst_p = jnp.concatenate([ei[1], N + lane]).reshape(NW, NCHUNKS, CHUNK)
    zeros_slab = jnp.zeros((ROWS_PER_TILE, H), jnp.bfloat16)

    h0, h_bf = _h0_call(x, lin0_w, lin0_b.reshape(1, H))
    h = h0
    for l in range(L):
        beta = float(np.log(THETA / (l + 1) + 1.0))
        agg2 = _spmm(h_bf, src_p, dst_p, zeros_slab)
        h, h_bf = _layer_call(beta, agg2, h0, conv_w[l], want_f32=(l == L - 1))

    w4t = lin1_w.reshape(H, H, C).transpose(0, 2, 1).reshape(
        H * C, H).astype(jnp.bfloat16)
    return _final_call(h, w4t, lin1_b.reshape(1, C))


# speedup vs baseline: 16.6397x; 1.0492x over previous
import functools

import numpy as np
import jax
import jax.numpy as jnp
from jax import lax
from jax.experimental import pallas as pl
from jax.experimental.pallas import tpu as pltpu
from jax.experimental.pallas import tpu_sc as plsc

N = 10000
E = 320000
D = 128
H = 64
C = 40
L = 4
ALPHA = 0.1
THETA = 0.5
MIN_NORM = 1e-15
EPS = 4e-3
MAXNORM = 1.0 - EPS

NC = 2
NS = 16
NW = NC * NS
CHUNK = 128
NCHUNKS = 80
E_PAD = NW * NCHUNKS * CHUNK
N_PAD = 12000
ROWS_PER_TILE = N_PAD // NS
CPAD = 128


NBUF = 8


def _spmm_body(h_hbm, srcp_hbm, dstp_hbm, zeros_hbm, agg_hbm,
               src_v, dst_v, rows_v, agg_sh, gsems, ssems):
    cid = lax.axis_index("c")
    sid = lax.axis_index("s")
    wid = cid * NS + sid

    pltpu.sync_copy(srcp_hbm.at[wid], src_v)
    pltpu.sync_copy(dstp_hbm.at[wid], dst_v)
    pltpu.sync_copy(zeros_hbm, agg_sh.at[pl.ds(sid * ROWS_PER_TILE, ROWS_PER_TILE)])
    plsc.subcore_barrier()

    def gather(j, b):
        return pltpu.make_async_copy(h_hbm.at[src_v.at[j]], rows_v.at[b], gsems[b])

    def scatter(j, b):
        return pltpu.make_async_copy(rows_v.at[b], agg_sh.at[dst_v.at[j]], ssems[b])

    for b in range(NBUF):
        pltpu.async_copy(h_hbm.at[src_v.at[b]], rows_v.at[b], gsems[b])
    for b in range(NBUF):
        gather(b, b).wait()
        pltpu.async_copy(rows_v.at[b], agg_sh.at[dst_v.at[b]], ssems[b], add=True)

    def body(it, carry):
        j0 = it * NBUF
        for b in range(NBUF):
            j = j0 + b
            scatter(j - NBUF, b).wait()
            pltpu.async_copy(h_hbm.at[src_v.at[j]], rows_v.at[b], gsems[b])
        for b in range(NBUF):
            j = j0 + b
            gather(j, b).wait()
            pltpu.async_copy(rows_v.at[b], agg_sh.at[dst_v.at[j]], ssems[b], add=True)
        return carry

    lax.fori_loop(1, NCHUNKS // NBUF, body, 0)
    for b in range(NBUF):
        scatter(NCHUNKS - NBUF + b, b).wait()
    plsc.subcore_barrier()
    out_base = cid * N_PAD + sid * ROWS_PER_TILE
    pltpu.sync_copy(agg_sh.at[pl.ds(sid * ROWS_PER_TILE, ROWS_PER_TILE)],
                    agg_hbm.at[pl.ds(out_base, ROWS_PER_TILE)])


_spmm = functools.partial(
    pl.kernel,
    out_type=jax.ShapeDtypeStruct((NC * N_PAD, H), jnp.bfloat16),
    mesh=plsc.VectorSubcoreMesh(core_axis_name="c", subcore_axis_name="s"),
    scratch_types=[
        pltpu.VMEM((NCHUNKS, CHUNK), jnp.int32),
        pltpu.VMEM((NCHUNKS, CHUNK), jnp.int32),
        pltpu.VMEM((NBUF, CHUNK, H), jnp.bfloat16),
        pltpu.VMEM_SHARED((N_PAD, H), jnp.bfloat16),
        [pltpu.SemaphoreType.DMA] * NBUF,
        [pltpu.SemaphoreType.DMA] * NBUF,
    ],
    compiler_params=pltpu.CompilerParams(use_tc_tiling_on_sc=False),
)(_spmm_body)


BN0 = 2000


def _h0_body(x_ref, w_ref, b_ref, o_ref, ob_ref):
    h = jnp.maximum(
        jnp.dot(x_ref[...], w_ref[...], preferred_element_type=jnp.float32)
        + b_ref[...], 0.0)
    o_ref[...] = h
    ob_ref[...] = h.astype(jnp.bfloat16)


def _h0_call(x, w, b):
    return pl.pallas_call(
        _h0_body,
        grid=(N // BN0,),
        in_specs=[
            pl.BlockSpec((BN0, D), lambda i: (i, 0)),
            pl.BlockSpec((D, H), lambda i: (0, 0)),
            pl.BlockSpec((1, H), lambda i: (0, 0)),
        ],
        out_specs=[
            pl.BlockSpec((BN0, H), lambda i: (i, 0)),
            pl.BlockSpec((BN0, H), lambda i: (i, 0)),
        ],
        out_shape=[
            jax.ShapeDtypeStruct((N, H), jnp.float32),
            jax.ShapeDtypeStruct((N, H), jnp.bfloat16),
        ],
    )(x, w, b)


def _layer_body(beta, want_f32, a0_ref, a1_ref, h0_ref, w_ref, *o_refs):
    agg = a0_ref[...].astype(jnp.float32) + a1_ref[...].astype(jnp.float32)
    hh = (1.0 - ALPHA) * agg + ALPHA * h0_ref[...]
    out = (1.0 - beta) * hh + beta * jnp.dot(
        hh, w_ref[...], preferred_element_type=jnp.float32)
    h = jnp.maximum(out, 0.0)
    if want_f32:
        o_refs[0][...] = h
        o_refs[1][...] = h.astype(jnp.bfloat16)
    else:
        o_refs[0][...] = h.astype(jnp.bfloat16)


def _layer_call(beta, agg2, h0, w, want_f32):
    nout = 2 if want_f32 else 1
    shapes = ([jax.ShapeDtypeStruct((N, H), jnp.float32)] if want_f32 else []) + [
        jax.ShapeDtypeStruct((N, H), jnp.bfloat16)]
    out = pl.pallas_call(
        functools.partial(_layer_body, beta, want_f32),
        grid=(N // BN0,),
        in_specs=[
            pl.BlockSpec((BN0, H), lambda i: (i, 0)),
            pl.BlockSpec((BN0, H), lambda i: (i + N_PAD // BN0, 0)),
            pl.BlockSpec((BN0, H), lambda i: (i, 0)),
            pl.BlockSpec((H, H), lambda i: (0, 0)),
        ],
        out_specs=[pl.BlockSpec((BN0, H), lambda i: (i, 0))] * nout,
        out_shape=shapes,
    )(agg2, agg2, h0, w)
    return out if want_f32 else (None, out[0])


BNF = 1000


def _final_body(h_ref, w4t_ref, b_ref, o_ref):
    h = h_ref[...]
    ht = jnp.transpose(h)
    tt = jnp.dot(w4t_ref[...], ht.astype(jnp.bfloat16),
                 preferred_element_type=jnp.float32)
    acc = jnp.zeros((C, BNF), jnp.float32)
    for i in range(H):
        acc = acc + ht[i:i + 1, :] * tt[i * C:(i + 1) * C, :]
    s1 = jnp.sum(ht * ht, axis=0, keepdims=True)
    nrm = jnp.maximum(s1, MIN_NORM)
    f1 = jnp.where(nrm > MAXNORM, MAXNORM / nrm, 1.0)
    pn = jnp.maximum(nrm * f1, MIN_NORM)
    pc = jnp.minimum(pn, 1.0 - 1e-7)
    art = 0.5 * (jnp.log1p(pc) - jnp.log1p(-pc))
    scale = f1 * art / pn
    y = scale * acc + jnp.transpose(b_ref[...])
    un = jnp.maximum(jnp.sqrt(jnp.sum(y * y, axis=0, keepdims=True)), MIN_NORM)
    res = jnp.tanh(un) * y / un
    rn = jnp.maximum(jnp.sqrt(jnp.sum(res * res, axis=0, keepdims=True)), MIN_NORM)
    res = jnp.where(rn > MAXNORM, res / rn * MAXNORM, res)
    m = jnp.max(res, axis=0, keepdims=True)
    z = res - m
    out_t = z - jnp.log(jnp.sum(jnp.exp(z), axis=0, keepdims=True))
    o_ref[...] = jnp.transpose(out_t)


def _final_call(h, w4t, b):
    return pl.pallas_call(
        _final_body,
        grid=(N // BNF,),
        in_specs=[
            pl.BlockSpec((BNF, H), lambda i: (i, 0)),
            pl.BlockSpec((H * C, H), lambda i: (0, 0)),
            pl.BlockSpec((1, C), lambda i: (0, 0)),
        ],
        out_specs=pl.BlockSpec((BNF, C), lambda i: (i, 0)),
        out_shape=jax.ShapeDtypeStruct((N, C), jnp.float32),
    )(h, w4t, b)


def kernel(x, edge_index, lin0_w, lin0_b, conv_w, lin1_w, lin1_b):
    x = x.astype(jnp.float32)
    ei = edge_index.astype(jnp.int32)
    lane = jnp.arange(E_PAD - E, dtype=jnp.int32) % CHUNK
    src_p = jnp.concatenate([ei[0], lane]).reshape(NW, NCHUNKS, CHUNK)
    dst_p = jnp.concatenate([ei[1], N + lane]).reshape(NW, NCHUNKS, CHUNK)
    zeros_slab = jnp.zeros((ROWS_PER_TILE, H), jnp.bfloat16)

    h0, h_bf = _h0_call(x, lin0_w, lin0_b.reshape(1, H))
    h = h0
    for l in range(L):
        beta = float(np.log(THETA / (l + 1) + 1.0))
        agg2 = _spmm(h_bf, src_p, dst_p, zeros_slab)
        h, h_bf = _layer_call(beta, agg2, h0, conv_w[l], want_f32=(l == L - 1))

    w4t = lin1_w.reshape(H, H, C).transpose(0, 2, 1).reshape(
        H * C, H).astype(jnp.bfloat16)
    return _final_call(h, w4t, lin1_b.reshape(1, C))
